# Initial kernel scaffold; baseline (speedup 1.0000x reference)
import jax
import jax.numpy as jnp
from jax.experimental import pallas as pl


def kernel(title_ids, token_ids, title_table, text_table):
    raise NotImplementedError("write your pallas kernel here")


# speedup vs baseline: 13.3514x; 13.3514x over previous
import functools

import jax
import jax.numpy as jnp
from jax import lax
from jax.experimental import pallas as pl
from jax.experimental.pallas import tpu as pltpu
from jax.experimental.pallas import tpu_sc as plsc

B = 16384
L = 20
D = 32
NC, NS = 2, 16
NW = NC * NS
CHUNK = B // NW
SUB = 128
NSUB = CHUNK // SUB
GATHER_W = 128


def _sc_body(ttl_hbm, tok_hbm, ttable_hbm, xtable_hbm, out_hbm,
             tok_idx, ttl_idx, rows, trows, packed, inv_buf, npad_buf,
             row0, sem):
    wid = lax.axis_index("s") * NC + lax.axis_index("c")
    base = wid * CHUNK

    pltpu.sync_copy(ttl_hbm.at[pl.ds(base, CHUNK)], ttl_idx)
    pltpu.sync_copy(tok_hbm.at[pl.ds(base * L, CHUNK * L)], tok_idx)
    pltpu.sync_copy(xtable_hbm.at[0], row0)
    r0a = row0[pl.ds(0, 16)]
    r0b = row0[pl.ds(16, 16)]
    lane = lax.iota(jnp.int32, 16)
    zero16 = jnp.zeros((16,), jnp.float32)

    for s in range(NSUB):
        sbase = s * SUB
        cps = [pltpu.async_copy(
            ttable_hbm.at[ttl_idx.at[pl.ds(sbase, SUB)]], trows, sem)]
        for j in range(SUB * L // GATHER_W):
            cps.append(pltpu.async_copy(
                xtable_hbm.at[tok_idx.at[pl.ds(sbase * L + j * GATHER_W,
                                               GATHER_W)]],
                rows.at[pl.ds(j * GATHER_W, GATHER_W)], sem))
        for c in cps:
            c.wait()

        def cbody(g, carry):
            tbase = (sbase + g * 16 + lane) * L
            cnt = zero16
            for l in range(L):
                ids = plsc.load_gather(tok_idx, [tbase + l])
                cnt = cnt + (ids != 0).astype(jnp.float32)
            inv_buf[pl.ds(g * 16, 16)] = (
                jnp.ones((16,), jnp.float32) / jnp.maximum(cnt, 1.0))
            npad_buf[pl.ds(g * 16, 16)] = (
                jnp.full((16,), float(L), jnp.float32) - cnt)
            return carry

        lax.fori_loop(0, SUB // 16, cbody, 0)

        def body(i, carry):
            i16 = jnp.broadcast_to(i, (16,))
            inv = plsc.load_gather(inv_buf, [i16])
            npad = plsc.load_gather(npad_buf, [i16])
            s0, s1 = zero16, zero16
            for l in range(L):
                s0 = s0 + rows[i * L + l, pl.ds(0, 16)]
                s1 = s1 + rows[i * L + l, pl.ds(16, 16)]
            packed[i, pl.ds(0, 16)] = trows[i, pl.ds(0, 16)]
            packed[i, pl.ds(16, 16)] = trows[i, pl.ds(16, 16)]
            packed[i, pl.ds(32, 16)] = (s0 - npad * r0a) * inv
            packed[i, pl.ds(48, 16)] = (s1 - npad * r0b) * inv
            return carry

        lax.fori_loop(0, SUB, body, 0)
        pltpu.sync_copy(packed, out_hbm.at[pl.ds(base + sbase, SUB), :])


@jax.jit
def _product_model(title_ids, token_ids_flat, title_table, text_table):
    mesh = plsc.VectorSubcoreMesh(core_axis_name="c", subcore_axis_name="s")
    f = functools.partial(
        pl.kernel,
        out_type=jax.ShapeDtypeStruct((B, 2 * D), jnp.float32),
        mesh=mesh,
        scratch_types=[
            pltpu.VMEM((CHUNK * L,), jnp.int32),
            pltpu.VMEM((CHUNK,), jnp.int32),
            pltpu.VMEM((SUB * L, D), jnp.float32),
            pltpu.VMEM((SUB, D), jnp.float32),
            pltpu.VMEM((SUB, 2 * D), jnp.float32),
            pltpu.VMEM((SUB,), jnp.float32),
            pltpu.VMEM((SUB,), jnp.float32),
            pltpu.VMEM((D,), jnp.float32),
            pltpu.SemaphoreType.DMA,
        ],
        compiler_params=pltpu.CompilerParams(use_tc_tiling_on_sc=False,
                                             needs_layout_passes=False),
    )(_sc_body)
    return f(title_ids, token_ids_flat, title_table, text_table)


def kernel(title_ids, token_ids, title_table, text_table):
    return _product_model(title_ids.astype(jnp.int32),
                          token_ids.reshape(-1).astype(jnp.int32),
                          title_table, text_table)
